# baseline (speedup 1.0000x reference)
import jax
import jax.numpy as jnp
from jax.experimental import pallas as pl
from jax.experimental.pallas import tpu as pltpu

N = 100000
CH = 128
BLK = 5000
NB = N // BLK


def _body(x_ref, seg_ref, u_ref, qw1_ref, qb1_ref, qw2_ref, qb2_ref,
          nw1_ref, nb1_ref, nw2_ref, nb2_ref, gw1_ref, gb1_ref, gw2r_ref,
          out_ref, q_s, m_s, s_s, acc_s):
    i = pl.program_id(0)

    @pl.when(i == 0)
    def _init():
        uq = jnp.maximum(
            jnp.dot(u_ref[:], qw1_ref[:], preferred_element_type=jnp.float32)
            + qb1_ref[:], 0.0)
        q_s[:] = (jnp.dot(uq, qw2_ref[:], preferred_element_type=jnp.float32)
                  + qb2_ref[:]).astype(jnp.bfloat16)
        m_s[:] = jnp.full((1, CH), -jnp.inf, jnp.float32)
        s_s[:] = jnp.zeros((1, CH), jnp.float32)
        acc_s[:] = jnp.zeros((CH, CH), jnp.float32)

    x = x_ref[:].astype(jnp.bfloat16)
    h1 = jnp.maximum(
        jnp.dot(x, nw1_ref[:].astype(jnp.bfloat16),
                preferred_element_type=jnp.float32)
        + nb1_ref[:], 0.0).astype(jnp.bfloat16)
    h = jnp.dot(h1, nw2_ref[:].astype(jnp.bfloat16),
                preferred_element_type=jnp.float32) + nb2_ref[:]

    seg = seg_ref[0]
    lane = jax.lax.broadcasted_iota(jnp.int32, (BLK, CH), 1)
    oh = seg == lane
    ohb = oh.astype(jnp.bfloat16)

    qg = jnp.dot(ohb, q_s[:], preferred_element_type=jnp.float32)
    gin = (qg * h).astype(jnp.bfloat16)
    g1 = jnp.maximum(
        jnp.dot(gin, gw1_ref[:].astype(jnp.bfloat16),
                preferred_element_type=jnp.float32)
        + gb1_ref[:], 0.0)
    gate = jnp.sum(g1 * gw2r_ref[:], axis=1, keepdims=True)

    bm = jnp.max(jnp.where(oh, gate, -jnp.inf), axis=0, keepdims=True)
    m_old = m_s[:]
    m_new = jnp.maximum(m_old, bm)
    scale = jnp.where(m_old == -jnp.inf, 0.0, jnp.exp(m_old - m_new))
    mg = jnp.sum(jnp.where(oh, m_new, 0.0), axis=1, keepdims=True)
    e = jnp.exp(gate - mg)
    ohe = jnp.where(oh, e, 0.0)

    s_s[:] = s_s[:] * scale + jnp.sum(ohe, axis=0, keepdims=True)
    acc_s[:] = acc_s[:] * scale + jax.lax.dot_general(
        h.astype(jnp.bfloat16), ohe.astype(jnp.bfloat16),
        (((0,), (0,)), ((), ())),
        preferred_element_type=jnp.float32)
    m_s[:] = m_new

    @pl.when(i == NB - 1)
    def _fin():
        out_ref[:] = acc_s[:] / (s_s[:] + 1e-16)


def kernel(x, u, batch, size, gate_w1, gate_b1, gate_w2, gate_b2,
           node_w1, node_b1, node_w2, node_b2,
           ques_w1, ques_b1, ques_w2, ques_b2):
    num_seg = u.shape[0]
    seg = batch.astype(jnp.int32) + (jnp.asarray(size, jnp.int32)
                                     - jnp.int32(num_seg))
    seg3 = seg.reshape(NB, BLK, 1)
    u_pad = jnp.zeros((CH, CH), jnp.float32).at[:num_seg].set(u)

    full = pl.BlockSpec((CH, CH), lambda i: (0, 0))
    row = pl.BlockSpec((1, CH), lambda i: (0, 0))
    accT = pl.pallas_call(
        _body,
        grid=(NB,),
        in_specs=[
            pl.BlockSpec((BLK, CH), lambda i: (i, 0)),
            pl.BlockSpec((1, BLK, 1), lambda i: (i, 0, 0)),
            full,
            full, row,
            full, row,
            full, row,
            full, row,
            full, row,
            row,
        ],
        out_specs=pl.BlockSpec((CH, CH), lambda i: (0, 0)),
        out_shape=jax.ShapeDtypeStruct((CH, CH), jnp.float32),
        scratch_shapes=[
            pltpu.VMEM((CH, CH), jnp.bfloat16),
            pltpu.VMEM((1, CH), jnp.float32),
            pltpu.VMEM((1, CH), jnp.float32),
            pltpu.VMEM((CH, CH), jnp.float32),
        ],
    )(x, seg3, u_pad,
      ques_w1, ques_b1.reshape(1, CH), ques_w2, ques_b2.reshape(1, CH),
      node_w1, node_b1.reshape(1, CH), node_w2, node_b2.reshape(1, CH),
      gate_w1, gate_b1.reshape(1, CH), gate_w2.reshape(1, CH))
    return accT[:, :num_seg].T


# speedup vs baseline: 1.1287x; 1.1287x over previous
import jax
import jax.numpy as jnp
from jax.experimental import pallas as pl
from jax.experimental.pallas import tpu as pltpu

N = 100000
CH = 128
BLK = 4000
NB = N // BLK


def _body(x_ref, seg_ref, u_ref, qw1_ref, qb1_ref, qw2_ref, qb2_ref,
          nw1_ref, nb1_ref, nw2_ref, nb2_ref, gw1_ref, gb1_ref, gw2r_ref,
          out_ref, q_s, m_s, s_s, acc_s):
    i = pl.program_id(0)

    @pl.when(i == 0)
    def _init():
        uq = jnp.maximum(
            jnp.dot(u_ref[:], qw1_ref[:], preferred_element_type=jnp.float32)
            + qb1_ref[:], 0.0)
        q_s[:] = (jnp.dot(uq, qw2_ref[:], preferred_element_type=jnp.float32)
                  + qb2_ref[:]).astype(jnp.bfloat16)
        m_s[:] = jnp.full((1, CH), -jnp.inf, jnp.float32)
        s_s[:] = jnp.zeros((1, CH), jnp.float32)
        acc_s[:] = jnp.zeros((CH, CH), jnp.float32)

    x = x_ref[:].astype(jnp.bfloat16)
    h1 = jnp.maximum(
        jnp.dot(x, nw1_ref[:].astype(jnp.bfloat16),
                preferred_element_type=jnp.float32)
        + nb1_ref[:], 0.0).astype(jnp.bfloat16)
    h = jnp.dot(h1, nw2_ref[:].astype(jnp.bfloat16),
                preferred_element_type=jnp.float32) + nb2_ref[:]

    seg = seg_ref[0]
    lane = jax.lax.broadcasted_iota(jnp.int32, (BLK, CH), 1)
    oh = seg == lane
    ohb = oh.astype(jnp.bfloat16)

    qg = jnp.dot(ohb, q_s[:], preferred_element_type=jnp.float32)
    gin = (qg * h).astype(jnp.bfloat16)
    g1 = jnp.maximum(
        jnp.dot(gin, gw1_ref[:].astype(jnp.bfloat16),
                preferred_element_type=jnp.float32)
        + gb1_ref[:], 0.0)
    gate = jnp.sum(g1 * gw2r_ref[:], axis=1, keepdims=True)

    bm = jnp.max(jnp.where(oh, gate, -jnp.inf), axis=0, keepdims=True)
    m_old = m_s[:]
    m_new = jnp.maximum(m_old, bm)
    scale = jnp.where(m_old == -jnp.inf, 0.0, jnp.exp(m_old - m_new))
    mg = jnp.sum(jnp.where(oh, m_new, 0.0), axis=1, keepdims=True)
    e = jnp.exp(gate - mg)
    ohe = jnp.where(oh, e, 0.0)

    s_s[:] = s_s[:] * scale + jnp.sum(ohe, axis=0, keepdims=True)
    acc_s[:] = acc_s[:] * scale + jax.lax.dot_general(
        h.astype(jnp.bfloat16), ohe.astype(jnp.bfloat16),
        (((0,), (0,)), ((), ())),
        preferred_element_type=jnp.float32)
    m_s[:] = m_new

    @pl.when(i == NB - 1)
    def _fin():
        out_ref[:] = acc_s[:] / (s_s[:] + 1e-16)


def kernel(x, u, batch, size, gate_w1, gate_b1, gate_w2, gate_b2,
           node_w1, node_b1, node_w2, node_b2,
           ques_w1, ques_b1, ques_w2, ques_b2):
    num_seg = u.shape[0]
    seg = batch.astype(jnp.int32) + (jnp.asarray(size, jnp.int32)
                                     - jnp.int32(num_seg))
    seg3 = seg.reshape(NB, BLK, 1)
    u_pad = jnp.zeros((CH, CH), jnp.float32).at[:num_seg].set(u)

    full = pl.BlockSpec((CH, CH), lambda i: (0, 0))
    row = pl.BlockSpec((1, CH), lambda i: (0, 0))
    accT = pl.pallas_call(
        _body,
        grid=(NB,),
        in_specs=[
            pl.BlockSpec((BLK, CH), lambda i: (i, 0)),
            pl.BlockSpec((1, BLK, 1), lambda i: (i, 0, 0)),
            full,
            full, row,
            full, row,
            full, row,
            full, row,
            full, row,
            row,
        ],
        out_specs=pl.BlockSpec((CH, CH), lambda i: (0, 0)),
        out_shape=jax.ShapeDtypeStruct((CH, CH), jnp.float32),
        scratch_shapes=[
            pltpu.VMEM((CH, CH), jnp.bfloat16),
            pltpu.VMEM((1, CH), jnp.float32),
            pltpu.VMEM((1, CH), jnp.float32),
            pltpu.VMEM((CH, CH), jnp.float32),
        ],
    )(x, seg3, u_pad,
      ques_w1, ques_b1.reshape(1, CH), ques_w2, ques_b2.reshape(1, CH),
      node_w1, node_b1.reshape(1, CH), node_w2, node_b2.reshape(1, CH),
      gate_w1, gate_b1.reshape(1, CH), gate_w2.reshape(1, CH))
    return accT[:, :num_seg].T
